# baseline (speedup 1.0000x reference)
import jax
import jax.numpy as jnp
from jax import lax
from jax.experimental import pallas as pl
from jax.experimental.pallas import tpu as pltpu
from jax.experimental.pallas import tpu_sc as plsc

BATCH = 16384
EMBED_DIM = 64
ROW_PAD = 128
NUM_NODES = 1000000
NUM_RELS = 1000
NUM_CORES = 2
NUM_SUBCORES = 16
NUM_WORKERS = NUM_CORES * NUM_SUBCORES
B_PER_W = BATCH // NUM_WORKERS
CHUNK = 128
CHUNKS_PER_W = B_PER_W // CHUNK
BVEC = 16

T_COLS = 1024
T_GRID = (NUM_NODES + T_COLS - 1) // T_COLS


def _transpose_tile(x_ref, o_ref):
    x = x_ref[...]
    o_ref[:, :EMBED_DIM] = jnp.transpose(x)


def _gather_body(node_hbm, rel_hbm, head_hbm, relids_hbm, tail_hbm, out_hbm,
                 idx_h, idx_r, idx_t, h_buf, r_buf, t_buf, o_buf,
                 sem_h, sem_r, sem_t):
    wid = lax.axis_index("s") * NUM_CORES + lax.axis_index("c")
    base = wid * B_PER_W
    idx_row = wid * CHUNKS_PER_W

    pltpu.sync_copy(head_hbm.at[pl.ds(idx_row, CHUNKS_PER_W)], idx_h)
    pltpu.sync_copy(relids_hbm.at[pl.ds(idx_row, CHUNKS_PER_W)], idx_r)
    pltpu.sync_copy(tail_hbm.at[pl.ds(idx_row, CHUNKS_PER_W)], idx_t)

    for c in range(CHUNKS_PER_W):
        ch = pltpu.async_copy(node_hbm.at[idx_h.at[c]], h_buf, sem_h)
        cr = pltpu.async_copy(rel_hbm.at[idx_r.at[c]], r_buf, sem_r)
        ct = pltpu.async_copy(node_hbm.at[idx_t.at[c]], t_buf, sem_t)
        ch.wait()
        cr.wait()
        ct.wait()

        def row_body(i, carry):
            for j in range(EMBED_DIM // BVEC):
                sl = pl.ds(j * BVEC, BVEC)
                o_buf[i, sl] = h_buf[i, sl] + r_buf[i, sl] + t_buf[i, sl]
            return carry

        lax.fori_loop(0, CHUNK, row_body, 0)

        pltpu.sync_copy(o_buf, out_hbm.at[pl.ds(base + c * CHUNK, CHUNK)])


@jax.jit
def kernel(head_ids, rel_ids, tail_ids, node_embeddings, rel_weight):
    k1 = pl.pallas_call(
        _transpose_tile,
        grid=(T_GRID,),
        in_specs=[pl.BlockSpec((EMBED_DIM, T_COLS), lambda i: (0, i))],
        out_specs=pl.BlockSpec((T_COLS, ROW_PAD), lambda i: (i, 0)),
        out_shape=jax.ShapeDtypeStruct((NUM_NODES, ROW_PAD), jnp.float32),
    )
    mesh = plsc.VectorSubcoreMesh(core_axis_name="c", subcore_axis_name="s",
                                  num_cores=NUM_CORES,
                                  num_subcores=NUM_SUBCORES)
    k2 = pl.kernel(
        _gather_body,
        out_type=jax.ShapeDtypeStruct((BATCH, ROW_PAD), jnp.float32),
        mesh=mesh,
        compiler_params=pltpu.CompilerParams(needs_layout_passes=False),
        scratch_types=[
            pltpu.VMEM((CHUNKS_PER_W, CHUNK), jnp.int32),
            pltpu.VMEM((CHUNKS_PER_W, CHUNK), jnp.int32),
            pltpu.VMEM((CHUNKS_PER_W, CHUNK), jnp.int32),
            pltpu.VMEM((CHUNK, ROW_PAD), jnp.float32),
            pltpu.VMEM((CHUNK, ROW_PAD), jnp.float32),
            pltpu.VMEM((CHUNK, ROW_PAD), jnp.float32),
            pltpu.VMEM((CHUNK, ROW_PAD), jnp.float32),
            pltpu.SemaphoreType.DMA,
            pltpu.SemaphoreType.DMA,
            pltpu.SemaphoreType.DMA,
        ],
    )
    node_bf = k1(node_embeddings.T)
    rel_bf = jnp.pad(rel_weight,
                     ((0, 0), (0, ROW_PAD - EMBED_DIM)))
    nrows = NUM_WORKERS * CHUNKS_PER_W
    head2d = head_ids.reshape(nrows, CHUNK)
    rel2d = rel_ids.reshape(nrows, CHUNK)
    tail2d = tail_ids.reshape(nrows, CHUNK)
    out_pad = k2(node_bf, rel_bf, head2d, rel2d, tail2d)
    return out_pad[:, :EMBED_DIM]
